# Initial kernel scaffold; baseline (speedup 1.0000x reference)
import jax
import jax.numpy as jnp
from jax.experimental import pallas as pl


def kernel(encoded, output_items, emb_table):
    raise NotImplementedError("write your pallas kernel here")


# speedup vs baseline: 1.7233x; 1.7233x over previous
import functools

import jax
import jax.numpy as jnp
from jax import lax
from jax.experimental import pallas as pl
from jax.experimental.pallas import tpu as pltpu
from jax.experimental.pallas import tpu_sc as plsc

LANES = 16
CHUNK = 104


def _make_sc_kernel(B, T, L, K, D, nc, ns, interpret=False):
    NW = nc * ns
    N = B * L * K
    per_b = L * K
    chunks_per_b = per_b // CHUNK
    b_per_w = B // NW
    n_chunks = b_per_w * chunks_per_b
    enc_rows = T * K
    enc_skip = (T - L) * K

    mesh = plsc.VectorSubcoreMesh(core_axis_name="c", subcore_axis_name="s",
                                  num_cores=nc, num_subcores=ns)

    @functools.partial(
        pl.kernel,
        out_type=jax.ShapeDtypeStruct((N,), jnp.float32),
        mesh=mesh,
        scratch_types=[
            pltpu.VMEM((CHUNK,), jnp.int32),
            pltpu.VMEM((CHUNK, D), jnp.float32),
            pltpu.VMEM((CHUNK * D,), jnp.float32),
            pltpu.VMEM((CHUNK,), jnp.float32),
            pltpu.SemaphoreType.DMA,
        ],
        compiler_params=pltpu.CompilerParams(
            needs_layout_passes=False,
            use_tc_tiling_on_sc=False,
        ),
        interpret=interpret,
    )
    def sc_kernel(enc_hbm, idx_hbm, table_hbm, out_hbm,
                  idx_v, rows_v, enc_v, out_v, sem):
        wid = lax.axis_index("s") * nc + lax.axis_index("c")
        lane = lax.iota(jnp.int32, LANES)
        last_lane = lane == (LANES - 1)

        def chunk_body(c, carry):
            b = wid * b_per_w + c // chunks_per_b
            r0 = (c % chunks_per_b) * CHUNK
            gbase = b * per_b + r0
            erow = b * enc_rows + enc_skip + r0
            pltpu.sync_copy(idx_hbm.at[pl.ds(gbase, CHUNK)], idx_v)
            gather = pltpu.async_copy(table_hbm.at[idx_v], rows_v, sem)
            pltpu.sync_copy(enc_hbm.at[pl.ds(erow * D, CHUNK * D)], enc_v)
            gather.wait()

            def item_body(i, carry2):
                acc = (rows_v[i, pl.ds(0, LANES)]
                       * enc_v[pl.ds(i * D, LANES)])
                for j in range(1, D // LANES):
                    acc = acc + (rows_v[i, pl.ds(j * LANES, LANES)]
                                 * enc_v[pl.ds(i * D + j * LANES, LANES)])
                cum = plsc.cumsum(acc)
                plsc.store_scatter(out_v, [jnp.full((LANES,), i, jnp.int32)],
                                   cum, mask=last_lane)
                return carry2

            lax.fori_loop(0, CHUNK, item_body, 0)
            pltpu.sync_copy(out_v, out_hbm.at[pl.ds(gbase, CHUNK)])
            return carry

        lax.fori_loop(0, n_chunks, chunk_body, 0)

    return sc_kernel


def kernel(encoded, output_items, emb_table):
    B, T, K, D = encoded.shape
    L = output_items.shape[1]
    info = plsc.get_sparse_core_info()
    k = _make_sc_kernel(B, T, L, K, D, info.num_cores, info.num_subcores)
    enc2 = encoded.reshape(-1)
    idx = output_items.reshape(-1).astype(jnp.int32)
    out = k(enc2, idx, emb_table)
    return out.reshape(B, L, K)
